# baseline (speedup 1.0000x reference)
import functools

import jax
import jax.numpy as jnp
import numpy as np
from jax import lax
from jax.experimental import pallas as pl
from jax.experimental.pallas import tpu as pltpu
from jax.experimental.pallas import tpu_sc as plsc

_BS = 32
_ROWS = 2 * _BS
_ROW_W = 160000
_C = 16000
_P = _ROW_W // _C
_NCHUNK = _ROWS * _P

_PERM = np.array([22, 18, 6, 26, 21, 27, 10, 20, 24, 4, 31, 14, 0, 3, 5, 17,
                  28, 2, 23, 1, 8, 16, 30, 7, 19, 15, 9, 13, 11, 25, 12, 29],
                 dtype=np.int64)

_S = 2
_PAD = 8
_NBUF = 4


def _chunk_map(nw: int) -> np.ndarray:
    src_row = np.concatenate([_PERM, np.arange(_BS, _ROWS)])
    out_chunk = np.arange(_NCHUNK)
    src_chunk = (src_row[out_chunk // _P] * _P + out_chunk % _P).astype(np.int32)
    k = _NCHUNK // nw
    nbatch = k // _S
    g = src_chunk.reshape(nw, nbatch, _S)
    pad = np.repeat(g[:, :, :1], _PAD - _S, axis=2)
    return np.concatenate([g, pad], axis=2)


def _build_sc_copy(nw: int):
    assert _NCHUNK % nw == 0
    k = _NCHUNK // nw
    nbatch = k // _S
    mesh = plsc.VectorSubcoreMesh(core_axis_name="c", subcore_axis_name="s")
    nc = plsc.get_sparse_core_info().num_cores

    @functools.partial(
        pl.kernel,
        mesh=mesh,
        out_type=jax.ShapeDtypeStruct((_NCHUNK, _C), jnp.float32),
        scratch_types=(
            [pltpu.VMEM((nbatch, _PAD), jnp.int32)]
            + [pltpu.VMEM((_S, _C), jnp.float32)] * _NBUF
            + [pltpu.SemaphoreType.DMA] * (2 * _NBUF)
        ),
    )
    def sc_copy(src, gmap, out, idx_v, *rest):
        bufs = rest[:_NBUF]
        gsems = rest[_NBUF:2 * _NBUF]
        wsems = rest[2 * _NBUF:]
        wid = lax.axis_index("s") * nc + lax.axis_index("c")
        base = wid * k
        pltpu.sync_copy(gmap.at[wid], idx_v)
        gets = [None] * nbatch
        puts = [None] * nbatch
        for j in range(nbatch):
            slot = j % _NBUF
            if j >= _NBUF:
                puts[j - _NBUF].wait()
            gets[j] = pltpu.async_copy(
                src.at[idx_v.at[j, pl.ds(0, _S)]], bufs[slot], gsems[slot])
            if j > 0:
                pj = j - 1
                gets[pj].wait()
                puts[pj] = pltpu.async_copy(
                    bufs[pj % _NBUF], out.at[pl.ds(base + pj * _S, _S)],
                    wsems[pj % _NBUF])
        last = nbatch - 1
        gets[last].wait()
        puts[last] = pltpu.async_copy(
            bufs[last % _NBUF], out.at[pl.ds(base + last * _S, _S)],
            wsems[last % _NBUF])
        for j in range(max(0, nbatch - _NBUF), nbatch):
            puts[j].wait()

    return sc_copy


def kernel(sources):
    info = plsc.get_sparse_core_info()
    nw = info.num_cores * info.num_subcores
    gmap = jnp.asarray(_chunk_map(nw))
    src = sources.reshape(_NCHUNK, _C)
    out = _build_sc_copy(nw)(src, gmap)
    return out.reshape(sources.shape)


# speedup vs baseline: 6.2394x; 6.2394x over previous
import functools

import jax
import jax.numpy as jnp
import numpy as np
from jax import lax
from jax.experimental import pallas as pl
from jax.experimental.pallas import tpu as pltpu
from jax.experimental.pallas import tpu_sc as plsc

_BS = 32
_ROWS = 2 * _BS
_ROW_W = 160000
_C = 16000
_P = _ROW_W // _C
_NCHUNK = _ROWS * _P

_PERM = np.array([22, 18, 6, 26, 21, 27, 10, 20, 24, 4, 31, 14, 0, 3, 5, 17,
                  28, 2, 23, 1, 8, 16, 30, 7, 19, 15, 9, 13, 11, 25, 12, 29],
                 dtype=np.int64)

_S = 2
_PAD = 8
_NBUF = 4


def _chunk_map(nw: int) -> np.ndarray:
    src_row = np.concatenate([_PERM, np.arange(_BS, _ROWS)])
    out_chunk = np.arange(_NCHUNK)
    src_chunk = (src_row[out_chunk // _P] * _P + out_chunk % _P).astype(np.int32)
    k = _NCHUNK // nw
    nbatch = k // _S
    g = src_chunk.reshape(nw, nbatch, _S)
    pad = np.repeat(g[:, :, :1], _PAD - _S, axis=2)
    return np.concatenate([g, pad], axis=2)


def _build_sc_copy(nw: int):
    assert _NCHUNK % nw == 0
    k = _NCHUNK // nw
    nbatch = k // _S
    mesh = plsc.VectorSubcoreMesh(core_axis_name="c", subcore_axis_name="s")
    nc = plsc.get_sparse_core_info().num_cores

    @functools.partial(
        pl.kernel,
        mesh=mesh,
        out_type=jax.ShapeDtypeStruct((_NCHUNK, 1, _C), jnp.float32),
        scratch_types=(
            [pltpu.VMEM((nbatch, _PAD), jnp.int32)]
            + [pltpu.VMEM((_S, 1, _C), jnp.float32)] * _NBUF
            + [pltpu.SemaphoreType.DMA] * (2 * _NBUF)
        ),
    )
    def sc_copy(src, gmap, out, idx_v, *rest):
        bufs = rest[:_NBUF]
        gsems = rest[_NBUF:2 * _NBUF]
        wsems = rest[2 * _NBUF:]
        wid = lax.axis_index("s") * nc + lax.axis_index("c")
        base = wid * k
        pltpu.sync_copy(gmap.at[wid], idx_v)
        gets = [None] * nbatch
        puts = [None] * nbatch
        for j in range(nbatch):
            slot = j % _NBUF
            if j >= _NBUF:
                puts[j - _NBUF].wait()
            gets[j] = pltpu.async_copy(
                src.at[idx_v.at[j, pl.ds(0, _S)]], bufs[slot], gsems[slot])
            if j > 0:
                pj = j - 1
                gets[pj].wait()
                puts[pj] = pltpu.async_copy(
                    bufs[pj % _NBUF], out.at[pl.ds(base + pj * _S, _S)],
                    wsems[pj % _NBUF])
        last = nbatch - 1
        gets[last].wait()
        puts[last] = pltpu.async_copy(
            bufs[last % _NBUF], out.at[pl.ds(base + last * _S, _S)],
            wsems[last % _NBUF])
        for j in range(max(0, nbatch - _NBUF), nbatch):
            puts[j].wait()

    return sc_copy


def kernel(sources):
    info = plsc.get_sparse_core_info()
    nw = info.num_cores * info.num_subcores
    gmap = jnp.asarray(_chunk_map(nw))
    src = sources.reshape(_NCHUNK, 1, _C)
    out = _build_sc_copy(nw)(src, gmap)
    return out.reshape(sources.shape)


# speedup vs baseline: 6.2563x; 1.0027x over previous
import functools

import jax
import jax.numpy as jnp
import numpy as np
from jax import lax
from jax.experimental import pallas as pl
from jax.experimental.pallas import tpu as pltpu
from jax.experimental.pallas import tpu_sc as plsc

_BS = 32
_ROWS = 2 * _BS
_ROW_W = 160000
_C = 32000
_P = _ROW_W // _C
_NCHUNK = _ROWS * _P

_PERM = np.array([22, 18, 6, 26, 21, 27, 10, 20, 24, 4, 31, 14, 0, 3, 5, 17,
                  28, 2, 23, 1, 8, 16, 30, 7, 19, 15, 9, 13, 11, 25, 12, 29],
                 dtype=np.int64)

_S = 1
_PAD = 8
_NBUF = 4


def _chunk_map(nw: int) -> np.ndarray:
    src_row = np.concatenate([_PERM, np.arange(_BS, _ROWS)])
    out_chunk = np.arange(_NCHUNK)
    src_chunk = (src_row[out_chunk // _P] * _P + out_chunk % _P).astype(np.int32)
    k = _NCHUNK // nw
    nbatch = k // _S
    g = src_chunk.reshape(nw, nbatch, _S)
    pad = np.repeat(g[:, :, :1], _PAD - _S, axis=2)
    return np.concatenate([g, pad], axis=2)


def _build_sc_copy(nw: int):
    assert _NCHUNK % nw == 0
    k = _NCHUNK // nw
    nbatch = k // _S
    mesh = plsc.VectorSubcoreMesh(core_axis_name="c", subcore_axis_name="s")
    nc = plsc.get_sparse_core_info().num_cores

    @functools.partial(
        pl.kernel,
        mesh=mesh,
        out_type=jax.ShapeDtypeStruct((_NCHUNK, 1, _C), jnp.float32),
        scratch_types=(
            [pltpu.VMEM((nbatch, _PAD), jnp.int32)]
            + [pltpu.VMEM((_S, 1, _C), jnp.float32)] * _NBUF
            + [pltpu.SemaphoreType.DMA] * (2 * _NBUF)
        ),
    )
    def sc_copy(src, gmap, out, idx_v, *rest):
        bufs = rest[:_NBUF]
        gsems = rest[_NBUF:2 * _NBUF]
        wsems = rest[2 * _NBUF:]
        wid = lax.axis_index("s") * nc + lax.axis_index("c")
        base = wid * k
        pltpu.sync_copy(gmap.at[wid], idx_v)
        gets = [None] * nbatch
        puts = [None] * nbatch
        for j in range(nbatch):
            slot = j % _NBUF
            if j >= _NBUF:
                puts[j - _NBUF].wait()
            gets[j] = pltpu.async_copy(
                src.at[idx_v.at[j, pl.ds(0, _S)]], bufs[slot], gsems[slot])
            if j > 0:
                pj = j - 1
                gets[pj].wait()
                puts[pj] = pltpu.async_copy(
                    bufs[pj % _NBUF], out.at[pl.ds(base + pj * _S, _S)],
                    wsems[pj % _NBUF])
        last = nbatch - 1
        gets[last].wait()
        puts[last] = pltpu.async_copy(
            bufs[last % _NBUF], out.at[pl.ds(base + last * _S, _S)],
            wsems[last % _NBUF])
        for j in range(max(0, nbatch - _NBUF), nbatch):
            puts[j].wait()

    return sc_copy


def kernel(sources):
    info = plsc.get_sparse_core_info()
    nw = info.num_cores * info.num_subcores
    gmap = jnp.asarray(_chunk_map(nw))
    src = sources.reshape(_NCHUNK, 1, _C)
    out = _build_sc_copy(nw)(src, gmap)
    return out.reshape(sources.shape)
